# baseline (speedup 1.0000x reference)
import functools

import jax
import jax.numpy as jnp
from jax import lax
from jax.experimental import pallas as pl
from jax.experimental.pallas import tpu as pltpu
from jax.experimental.pallas import tpu_sc as plsc

N = 10000
E = 160000
D = 256
H = 256
L = 3

NC = 2
NS = 16
HALF = H // NC
TPS = E // NS
CH = 40
NCHUNK = TPS // CH
NBUF = 7
NBLK = NCHUNK // NBUF
NREM = NCHUNK - NBLK * NBUF
BE = NBUF * CH
RPT = 624
REM_ROWS = N - NS * RPT

BN = 1000


def _tc_first_body(x_ref, wr_ref, wc_ref, b_ref, hr_ref, root_ref):
  h = x_ref[...]
  hr_ref[...] = jnp.dot(h, wr_ref[...], preferred_element_type=jnp.float32)
  root_ref[...] = (
      jnp.dot(h, wc_ref[...], preferred_element_type=jnp.float32) + b_ref[...])


def _tc_step_body(a0_ref, a1_ref, root_ref, wr_ref, wc_ref, b_ref,
                  hr_ref, rootn_ref):
  agg = jnp.concatenate([a0_ref[...], a1_ref[...]], axis=1)
  h = jnp.maximum(agg + root_ref[...], 0.0)
  hr_ref[...] = jnp.dot(h, wr_ref[...], preferred_element_type=jnp.float32)
  rootn_ref[...] = (
      jnp.dot(h, wc_ref[...], preferred_element_type=jnp.float32) + b_ref[...])


def _tc_final_body(a0_ref, a1_ref, root_ref, out_ref):
  agg = jnp.concatenate([a0_ref[...], a1_ref[...]], axis=1)
  out_ref[...] = jnp.maximum(agg + root_ref[...], 0.0)


def _row_spec(w):
  return pl.BlockSpec((BN, w), lambda i: (i, 0))


def _full_spec(h, w):
  return pl.BlockSpec((h, w), lambda i: (0, 0))


_tc_first = pl.pallas_call(
    _tc_first_body,
    grid=(N // BN,),
    in_specs=[_row_spec(D), _full_spec(D, H), _full_spec(D, H),
              _full_spec(1, H)],
    out_specs=[_row_spec(H), _row_spec(H)],
    out_shape=[jax.ShapeDtypeStruct((N, H), jnp.float32),
               jax.ShapeDtypeStruct((N, H), jnp.float32)],
)

_tc_step = pl.pallas_call(
    _tc_step_body,
    grid=(N // BN,),
    in_specs=[_row_spec(HALF), _row_spec(HALF), _row_spec(H),
              _full_spec(H, H), _full_spec(H, H), _full_spec(1, H)],
    out_specs=[_row_spec(H), _row_spec(H)],
    out_shape=[jax.ShapeDtypeStruct((N, H), jnp.float32),
               jax.ShapeDtypeStruct((N, H), jnp.float32)],
)

_tc_final = pl.pallas_call(
    _tc_final_body,
    grid=(N // BN,),
    in_specs=[_row_spec(HALF), _row_spec(HALF), _row_spec(H)],
    out_specs=_row_spec(H),
    out_shape=jax.ShapeDtypeStruct((N, H), jnp.float32),
)


_mesh = plsc.VectorSubcoreMesh(
    core_axis_name="c", subcore_axis_name="s", num_cores=NC, num_subcores=NS)


@functools.partial(
    pl.kernel,
    out_type=[jax.ShapeDtypeStruct((N, HALF), jnp.float32),
              jax.ShapeDtypeStruct((N, HALF), jnp.float32)],
    mesh=_mesh,
    scratch_types=(
        [pltpu.VMEM_SHARED((N, HALF), jnp.float32)]
        + [pltpu.VMEM((TPS,), jnp.float32)]
        + [pltpu.VMEM((BE, HALF), jnp.float32)]
        + [pltpu.VMEM((CH,), jnp.int32)] * NBUF
        + [pltpu.VMEM((CH,), jnp.int32)] * NBUF
        + [pltpu.SemaphoreType.DMA] * (3 * NBUF)
    ),
)
def _sc_agg(hr2_hbm, idx2_hbm, dst_hbm, w_hbm, out0_hbm, out1_hbm,
            acc, w_all, rows, *rest):
  idxb = rest[:NBUF]
  dstb = rest[NBUF:2 * NBUF]
  gsems = rest[2 * NBUF:3 * NBUF]
  ssems = rest[3 * NBUF:4 * NBUF]
  psems = rest[4 * NBUF:5 * NBUF]
  c = lax.axis_index("c")
  s = lax.axis_index("s")

  zero = jnp.zeros((16,), jnp.float32)

  def zrow(i, carry):
    for j in range(HALF // 16):
      rows[i, pl.ds(j * 16, 16)] = zero
    return carry

  lax.fori_loop(0, CH, zrow, 0)

  base = s * RPT
  nfull = RPT // CH
  for k in range(nfull):
    pltpu.sync_copy(rows.at[pl.ds(0, CH)], acc.at[pl.ds(base + k * CH, CH)])
  rem = RPT - nfull * CH
  pltpu.sync_copy(rows.at[pl.ds(0, rem)],
                  acc.at[pl.ds(base + nfull * CH, rem)])

  @pl.when(s == NS - 1)
  def _():
    pltpu.sync_copy(rows.at[pl.ds(0, REM_ROWS)],
                    acc.at[pl.ds(NS * RPT, REM_ROWS)])

  pltpu.sync_copy(w_hbm.at[pl.ds(s * TPS, TPS)], w_all)

  plsc.subcore_barrier()

  def start_params(k, i):
    e0 = s * TPS + i * CH
    pltpu.async_copy(idx2_hbm.at[pl.ds(c * E + e0, CH)], idxb[k], psems[k])
    pltpu.async_copy(dst_hbm.at[pl.ds(e0, CH)], dstb[k], psems[k])

  def wait_params(k):
    pltpu.make_async_copy(idx2_hbm.at[pl.ds(0, CH)], idxb[k], psems[k]).wait()
    pltpu.make_async_copy(dst_hbm.at[pl.ds(0, CH)], dstb[k], psems[k]).wait()

  def slot(k):
    return rows.at[pl.ds(k * CH, CH)]

  def start_gather(k):
    pltpu.async_copy(hr2_hbm.at[idxb[k]], slot(k), gsems[k])

  def wait_gather(k):
    pltpu.make_async_copy(hr2_hbm.at[idxb[k]], slot(k), gsems[k]).wait()

  def start_scatter(k):
    pltpu.async_copy(slot(k), acc.at[dstb[k]], ssems[k], add=True)

  def wait_scatter(k):
    pltpu.make_async_copy(slot(k), acc.at[dstb[k]], ssems[k]).wait()

  def scale_span(i0, nedge):
    def body(g, carry):
      wvec = w_all[pl.ds(i0 * CH + g * 16, 16)]
      for u in range(16):
        e = g * 16 + u
        wv = wvec[u]
        for j in range(HALF // 16):
          sl = pl.ds(j * 16, 16)
          rows[e, sl] = rows[e, sl] * wv
      return carry

    lax.fori_loop(0, nedge // 16, body, 0)
    ntail = nedge % 16
    if ntail:
      wvec = w_all[pl.ds(i0 * CH + nedge - 16, 16)]
      for u in range(ntail):
        e = nedge - ntail + u
        wv = wvec[16 - ntail + u]
        for j in range(HALF // 16):
          sl = pl.ds(j * 16, 16)
          rows[e, sl] = rows[e, sl] * wv

  for k in range(NBUF):
    start_params(k, k)
  for k in range(NBUF):
    wait_params(k)
    start_gather(k)

  def gblock(g, carry):
    for k in range(NBUF):
      wait_gather(k)
    scale_span(g * NBUF, BE)
    for k in range(NBUF):
      start_scatter(k)
    for k in range(NBUF):
      wait_scatter(k)

      @pl.when(g < NBLK - 1)
      def _(k=k):
        start_params(k, (g + 1) * NBUF + k)
    for k in range(NBUF):

      @pl.when(g < NBLK - 1)
      def _(k=k):
        wait_params(k)
        start_gather(k)
    return carry

  lax.fori_loop(0, NBLK, gblock, 0)

  if NREM:
    for r in range(NREM):
      start_params(r, NBLK * NBUF + r)
    for r in range(NREM):
      wait_params(r)
      start_gather(r)
    for r in range(NREM):
      wait_gather(r)
    scale_span(NBLK * NBUF, NREM * CH)
    for r in range(NREM):
      start_scatter(r)
    for r in range(NREM):
      wait_scatter(r)

  plsc.subcore_barrier()

  @pl.when(c == 0)
  def _():
    pltpu.sync_copy(acc.at[pl.ds(base, RPT)], out0_hbm.at[pl.ds(base, RPT)])

    @pl.when(s == NS - 1)
    def _():
      pltpu.sync_copy(acc.at[pl.ds(NS * RPT, REM_ROWS)],
                      out0_hbm.at[pl.ds(NS * RPT, REM_ROWS)])

  @pl.when(c == 1)
  def _():
    pltpu.sync_copy(acc.at[pl.ds(base, RPT)], out1_hbm.at[pl.ds(base, RPT)])

    @pl.when(s == NS - 1)
    def _():
      pltpu.sync_copy(acc.at[pl.ds(NS * RPT, REM_ROWS)],
                      out1_hbm.at[pl.ds(NS * RPT, REM_ROWS)])


@jax.jit
def kernel(x, edge_idx, edge_attr, W_rel, W_root, b):
  src = edge_idx[0]
  dst = edge_idx[1]
  idx2 = jnp.concatenate([2 * src, 2 * src + 1])
  b2 = b.reshape(L, 1, H)

  hr, root = _tc_first(x, W_rel[0], W_root[0], b2[0])
  for l in range(L):
    hr2 = hr.reshape(2 * N, HALF)
    a0, a1 = _sc_agg(hr2, idx2, dst, edge_attr)
    if l < L - 1:
      hr, root = _tc_step(a0, a1, root, W_rel[l + 1], W_root[l + 1], b2[l + 1])
    else:
      out = _tc_final(a0, a1, root)
  return out


# speedup vs baseline: 1.4746x; 1.4746x over previous
import functools

import jax
import jax.numpy as jnp
from jax import lax
from jax.experimental import pallas as pl
from jax.experimental.pallas import tpu as pltpu
from jax.experimental.pallas import tpu_sc as plsc

N = 10000
E = 160000
D = 256
H = 256
L = 3

NC = 2
NS = 16
HALF = H // NC
TPS = E // NS
CH = 40
NCHUNK = TPS // CH
NBUF = 5
NBLK = NCHUNK // NBUF
BE = NBUF * CH
LAG = 2
RPT = 624
REM_ROWS = N - NS * RPT

BN = 1000


def _tc_first_body(x_ref, wr_ref, wc_ref, b_ref, hr_ref, root_ref):
  h = x_ref[...]
  hr_ref[...] = jnp.dot(h, wr_ref[...], preferred_element_type=jnp.float32)
  root_ref[...] = (
      jnp.dot(h, wc_ref[...], preferred_element_type=jnp.float32) + b_ref[...])


def _tc_step_body(a0_ref, a1_ref, root_ref, wr_ref, wc_ref, b_ref,
                  hr_ref, rootn_ref):
  agg = jnp.concatenate([a0_ref[...], a1_ref[...]], axis=1)
  h = jnp.maximum(agg + root_ref[...], 0.0)
  hr_ref[...] = jnp.dot(h, wr_ref[...], preferred_element_type=jnp.float32)
  rootn_ref[...] = (
      jnp.dot(h, wc_ref[...], preferred_element_type=jnp.float32) + b_ref[...])


def _tc_final_body(a0_ref, a1_ref, root_ref, out_ref):
  agg = jnp.concatenate([a0_ref[...], a1_ref[...]], axis=1)
  out_ref[...] = jnp.maximum(agg + root_ref[...], 0.0)


def _row_spec(w):
  return pl.BlockSpec((BN, w), lambda i: (i, 0))


def _full_spec(h, w):
  return pl.BlockSpec((h, w), lambda i: (0, 0))


_tc_first = pl.pallas_call(
    _tc_first_body,
    grid=(N // BN,),
    in_specs=[_row_spec(D), _full_spec(D, H), _full_spec(D, H),
              _full_spec(1, H)],
    out_specs=[_row_spec(H), _row_spec(H)],
    out_shape=[jax.ShapeDtypeStruct((N, H), jnp.float32),
               jax.ShapeDtypeStruct((N, H), jnp.float32)],
)

_tc_step = pl.pallas_call(
    _tc_step_body,
    grid=(N // BN,),
    in_specs=[_row_spec(HALF), _row_spec(HALF), _row_spec(H),
              _full_spec(H, H), _full_spec(H, H), _full_spec(1, H)],
    out_specs=[_row_spec(H), _row_spec(H)],
    out_shape=[jax.ShapeDtypeStruct((N, H), jnp.float32),
               jax.ShapeDtypeStruct((N, H), jnp.float32)],
)

_tc_final = pl.pallas_call(
    _tc_final_body,
    grid=(N // BN,),
    in_specs=[_row_spec(HALF), _row_spec(HALF), _row_spec(H)],
    out_specs=_row_spec(H),
    out_shape=jax.ShapeDtypeStruct((N, H), jnp.float32),
)


_mesh = plsc.VectorSubcoreMesh(
    core_axis_name="c", subcore_axis_name="s", num_cores=NC, num_subcores=NS)


@functools.partial(
    pl.kernel,
    out_type=[jax.ShapeDtypeStruct((N, HALF), jnp.float32),
              jax.ShapeDtypeStruct((N, HALF), jnp.float32)],
    mesh=_mesh,
    scratch_types=(
        [pltpu.VMEM_SHARED((N, HALF), jnp.float32)]
        + [pltpu.VMEM((TPS,), jnp.int32)]
        + [pltpu.VMEM((TPS,), jnp.float32)]
        + [pltpu.VMEM((BE, HALF), jnp.float32)]
        + [pltpu.VMEM((CH,), jnp.int32)] * NBUF
        + [pltpu.SemaphoreType.DMA] * (3 * NBUF)
    ),
)
def _sc_agg(hr2_hbm, idx2_hbm, dst_hbm, w_hbm, out0_hbm, out1_hbm,
            acc, idx_all, w_all, rows, *rest):
  dstb = rest[:NBUF]
  gsems = rest[NBUF:2 * NBUF]
  ssems = rest[2 * NBUF:3 * NBUF]
  psems = rest[3 * NBUF:4 * NBUF]
  c = lax.axis_index("c")
  s = lax.axis_index("s")

  zero = jnp.zeros((16,), jnp.float32)

  def zrow(i, carry):
    for j in range(HALF // 16):
      rows[i, pl.ds(j * 16, 16)] = zero
    return carry

  lax.fori_loop(0, CH, zrow, 0)

  base = s * RPT
  nfull = RPT // CH
  for k in range(nfull):
    pltpu.sync_copy(rows.at[pl.ds(0, CH)], acc.at[pl.ds(base + k * CH, CH)])
  rem = RPT - nfull * CH
  pltpu.sync_copy(rows.at[pl.ds(0, rem)],
                  acc.at[pl.ds(base + nfull * CH, rem)])

  @pl.when(s == NS - 1)
  def _():
    pltpu.sync_copy(rows.at[pl.ds(0, REM_ROWS)],
                    acc.at[pl.ds(NS * RPT, REM_ROWS)])

  pltpu.sync_copy(idx2_hbm.at[pl.ds(c * E + s * TPS, TPS)], idx_all)
  pltpu.sync_copy(w_hbm.at[pl.ds(s * TPS, TPS)], w_all)

  plsc.subcore_barrier()

  def start_params(k, i):
    pltpu.async_copy(dst_hbm.at[pl.ds(s * TPS + i * CH, CH)], dstb[k],
                     psems[k])

  def wait_params(k):
    pltpu.make_async_copy(dst_hbm.at[pl.ds(0, CH)], dstb[k], psems[k]).wait()

  def slot(k):
    return rows.at[pl.ds(k * CH, CH)]

  def start_gather(k, i):
    pltpu.async_copy(hr2_hbm.at[idx_all.at[pl.ds(i * CH, CH)]],
                     slot(k), gsems[k])

  def wait_gather(k):
    pltpu.make_async_copy(hr2_hbm.at[idx_all.at[pl.ds(0, CH)]],
                          slot(k), gsems[k]).wait()

  def start_scatter(k):
    pltpu.async_copy(slot(k), acc.at[dstb[k]], ssems[k], add=True)

  def wait_scatter(k):
    pltpu.make_async_copy(slot(k), acc.at[dstb[k]], ssems[k]).wait()

  def scale_chunk(k, i):
    r0 = k * CH

    def body(g, carry):
      wvec = w_all[pl.ds(i * CH + g * 16, 16)]
      for u in range(16):
        e = r0 + g * 16 + u
        wv = wvec[u]
        for j in range(HALF // 16):
          sl = pl.ds(j * 16, 16)
          rows[e, sl] = rows[e, sl] * wv
      return carry

    lax.fori_loop(0, CH // 16, body, 0)
    ntail = CH % 16
    if ntail:
      wvec = w_all[pl.ds(i * CH + CH - 16, 16)]
      for u in range(ntail):
        e = r0 + CH - ntail + u
        wv = wvec[16 - ntail + u]
        for j in range(HALF // 16):
          sl = pl.ds(j * 16, 16)
          rows[e, sl] = rows[e, sl] * wv

  def drain_slot(j, g):
    wait_scatter(j)

    @pl.when(g < NBLK - 1)
    def _():
      i2 = (g + 1) * NBUF + j
      start_gather(j, i2)
      start_params(j, i2)

  for k in range(NBUF):
    start_params(k, k)
    start_gather(k, k)

  def gblock(g, carry):
    for k in range(NBUF):
      i = g * NBUF + k
      wait_gather(k)
      wait_params(k)
      scale_chunk(k, i)
      start_scatter(k)
      if k >= LAG:
        drain_slot(k - LAG, g)
    for j in range(NBUF - LAG, NBUF):
      drain_slot(j, g)
    return carry

  lax.fori_loop(0, NBLK, gblock, 0)

  plsc.subcore_barrier()

  @pl.when(c == 0)
  def _():
    pltpu.sync_copy(acc.at[pl.ds(base, RPT)], out0_hbm.at[pl.ds(base, RPT)])

    @pl.when(s == NS - 1)
    def _():
      pltpu.sync_copy(acc.at[pl.ds(NS * RPT, REM_ROWS)],
                      out0_hbm.at[pl.ds(NS * RPT, REM_ROWS)])

  @pl.when(c == 1)
  def _():
    pltpu.sync_copy(acc.at[pl.ds(base, RPT)], out1_hbm.at[pl.ds(base, RPT)])

    @pl.when(s == NS - 1)
    def _():
      pltpu.sync_copy(acc.at[pl.ds(NS * RPT, REM_ROWS)],
                      out1_hbm.at[pl.ds(NS * RPT, REM_ROWS)])


@jax.jit
def kernel(x, edge_idx, edge_attr, W_rel, W_root, b):
  src = edge_idx[0]
  dst = edge_idx[1]
  idx2 = jnp.concatenate([2 * src, 2 * src + 1])
  b2 = b.reshape(L, 1, H)

  hr, root = _tc_first(x, W_rel[0], W_root[0], b2[0])
  for l in range(L):
    hr2 = hr.reshape(2 * N, HALF)
    a0, a1 = _sc_agg(hr2, idx2, dst, edge_attr)
    if l < L - 1:
      hr, root = _tc_step(a0, a1, root, W_rel[l + 1], W_root[l + 1], b2[l + 1])
    else:
      out = _tc_final(a0, a1, root)
  return out


# speedup vs baseline: 1.5320x; 1.0389x over previous
import functools

import jax
import jax.numpy as jnp
from jax import lax
from jax.experimental import pallas as pl
from jax.experimental.pallas import tpu as pltpu
from jax.experimental.pallas import tpu_sc as plsc

N = 10000
E = 160000
D = 256
H = 256
L = 3

NC = 2
NS = 16
HALF = H // NC
TPS = E // NS
CH = 40
NCHUNK = TPS // CH
NBUF = 5
NBLK = NCHUNK // NBUF
BE = NBUF * CH
LAG = 1
RPT = 624
REM_ROWS = N - NS * RPT

BN = 1000


def _tc_first_body(x_ref, wr_ref, wc_ref, b_ref, hr_ref, root_ref):
  h = x_ref[...]
  hr_ref[...] = jnp.dot(h, wr_ref[...], preferred_element_type=jnp.float32)
  root_ref[...] = (
      jnp.dot(h, wc_ref[...], preferred_element_type=jnp.float32) + b_ref[...])


def _tc_step_body(a0_ref, a1_ref, root_ref, wr_ref, wc_ref, b_ref,
                  hr_ref, rootn_ref):
  agg = jnp.concatenate([a0_ref[...], a1_ref[...]], axis=1)
  h = jnp.maximum(agg + root_ref[...], 0.0)
  hr_ref[...] = jnp.dot(h, wr_ref[...], preferred_element_type=jnp.float32)
  rootn_ref[...] = (
      jnp.dot(h, wc_ref[...], preferred_element_type=jnp.float32) + b_ref[...])


def _tc_final_body(a0_ref, a1_ref, root_ref, out_ref):
  agg = jnp.concatenate([a0_ref[...], a1_ref[...]], axis=1)
  out_ref[...] = jnp.maximum(agg + root_ref[...], 0.0)


def _row_spec(w):
  return pl.BlockSpec((BN, w), lambda i: (i, 0))


def _full_spec(h, w):
  return pl.BlockSpec((h, w), lambda i: (0, 0))


_tc_first = pl.pallas_call(
    _tc_first_body,
    grid=(N // BN,),
    in_specs=[_row_spec(D), _full_spec(D, H), _full_spec(D, H),
              _full_spec(1, H)],
    out_specs=[_row_spec(H), _row_spec(H)],
    out_shape=[jax.ShapeDtypeStruct((N, H), jnp.float32),
               jax.ShapeDtypeStruct((N, H), jnp.float32)],
)

_tc_step = pl.pallas_call(
    _tc_step_body,
    grid=(N // BN,),
    in_specs=[_row_spec(HALF), _row_spec(HALF), _row_spec(H),
              _full_spec(H, H), _full_spec(H, H), _full_spec(1, H)],
    out_specs=[_row_spec(H), _row_spec(H)],
    out_shape=[jax.ShapeDtypeStruct((N, H), jnp.float32),
               jax.ShapeDtypeStruct((N, H), jnp.float32)],
)

_tc_final = pl.pallas_call(
    _tc_final_body,
    grid=(N // BN,),
    in_specs=[_row_spec(HALF), _row_spec(HALF), _row_spec(H)],
    out_specs=_row_spec(H),
    out_shape=jax.ShapeDtypeStruct((N, H), jnp.float32),
)


_mesh = plsc.VectorSubcoreMesh(
    core_axis_name="c", subcore_axis_name="s", num_cores=NC, num_subcores=NS)


@functools.partial(
    pl.kernel,
    out_type=[jax.ShapeDtypeStruct((N, HALF), jnp.float32),
              jax.ShapeDtypeStruct((N, HALF), jnp.float32)],
    mesh=_mesh,
    scratch_types=(
        [pltpu.VMEM_SHARED((N, HALF), jnp.float32)]
        + [pltpu.VMEM((TPS,), jnp.int32)]
        + [pltpu.VMEM((TPS,), jnp.float32)]
        + [pltpu.VMEM((BE, HALF), jnp.float32)]
        + [pltpu.VMEM((CH,), jnp.int32)] * NBUF
        + [pltpu.SemaphoreType.DMA] * (3 * NBUF)
    ),
)
def _sc_agg(hr2_hbm, idx2_hbm, dst_hbm, w_hbm, out0_hbm, out1_hbm,
            acc, idx_all, w_all, rows, *rest):
  dstb = rest[:NBUF]
  gsems = rest[NBUF:2 * NBUF]
  ssems = rest[2 * NBUF:3 * NBUF]
  psems = rest[3 * NBUF:4 * NBUF]
  c = lax.axis_index("c")
  s = lax.axis_index("s")

  zero = jnp.zeros((16,), jnp.float32)

  def zrow(i, carry):
    for j in range(HALF // 16):
      rows[i, pl.ds(j * 16, 16)] = zero
    return carry

  lax.fori_loop(0, CH, zrow, 0)

  base = s * RPT
  nfull = RPT // CH
  for k in range(nfull):
    pltpu.sync_copy(rows.at[pl.ds(0, CH)], acc.at[pl.ds(base + k * CH, CH)])
  rem = RPT - nfull * CH
  pltpu.sync_copy(rows.at[pl.ds(0, rem)],
                  acc.at[pl.ds(base + nfull * CH, rem)])

  @pl.when(s == NS - 1)
  def _():
    pltpu.sync_copy(rows.at[pl.ds(0, REM_ROWS)],
                    acc.at[pl.ds(NS * RPT, REM_ROWS)])

  pltpu.sync_copy(idx2_hbm.at[pl.ds(c * E + s * TPS, TPS)], idx_all)
  pltpu.sync_copy(w_hbm.at[pl.ds(s * TPS, TPS)], w_all)

  plsc.subcore_barrier()

  def start_params(k, i):
    pltpu.async_copy(dst_hbm.at[pl.ds(s * TPS + i * CH, CH)], dstb[k],
                     psems[k])

  def wait_params(k):
    pltpu.make_async_copy(dst_hbm.at[pl.ds(0, CH)], dstb[k], psems[k]).wait()

  def slot(k):
    return rows.at[pl.ds(k * CH, CH)]

  def start_gather(k, i):
    pltpu.async_copy(hr2_hbm.at[idx_all.at[pl.ds(i * CH, CH)]],
                     slot(k), gsems[k])

  def wait_gather(k):
    pltpu.make_async_copy(hr2_hbm.at[idx_all.at[pl.ds(0, CH)]],
                          slot(k), gsems[k]).wait()

  def start_scatter(k):
    pltpu.async_copy(slot(k), acc.at[dstb[k]], ssems[k], add=True)

  def wait_scatter(k):
    pltpu.make_async_copy(slot(k), acc.at[dstb[k]], ssems[k]).wait()

  def scale_chunk(k, i):
    r0 = k * CH

    def body(g, carry):
      wvec = w_all[pl.ds(i * CH + g * 16, 16)]
      for u in range(16):
        e = r0 + g * 16 + u
        wv = wvec[u]
        for j in range(HALF // 16):
          sl = pl.ds(j * 16, 16)
          rows[e, sl] = rows[e, sl] * wv
      return carry

    lax.fori_loop(0, CH // 16, body, 0)
    ntail = CH % 16
    if ntail:
      wvec = w_all[pl.ds(i * CH + CH - 16, 16)]
      for u in range(ntail):
        e = r0 + CH - ntail + u
        wv = wvec[16 - ntail + u]
        for j in range(HALF // 16):
          sl = pl.ds(j * 16, 16)
          rows[e, sl] = rows[e, sl] * wv

  def drain_slot(j, g):
    wait_scatter(j)

    @pl.when(g < NBLK - 1)
    def _():
      i2 = (g + 1) * NBUF + j
      start_gather(j, i2)
      start_params(j, i2)

  for k in range(NBUF):
    start_params(k, k)
    start_gather(k, k)

  def gblock(g, carry):
    for k in range(NBUF):
      i = g * NBUF + k
      wait_gather(k)
      wait_params(k)
      scale_chunk(k, i)
      start_scatter(k)
      if k >= LAG:
        drain_slot(k - LAG, g)
    for j in range(NBUF - LAG, NBUF):
      drain_slot(j, g)
    return carry

  lax.fori_loop(0, NBLK, gblock, 0)

  plsc.subcore_barrier()

  @pl.when(c == 0)
  def _():
    pltpu.sync_copy(acc.at[pl.ds(base, RPT)], out0_hbm.at[pl.ds(base, RPT)])

    @pl.when(s == NS - 1)
    def _():
      pltpu.sync_copy(acc.at[pl.ds(NS * RPT, REM_ROWS)],
                      out0_hbm.at[pl.ds(NS * RPT, REM_ROWS)])

  @pl.when(c == 1)
  def _():
    pltpu.sync_copy(acc.at[pl.ds(base, RPT)], out1_hbm.at[pl.ds(base, RPT)])

    @pl.when(s == NS - 1)
    def _():
      pltpu.sync_copy(acc.at[pl.ds(NS * RPT, REM_ROWS)],
                      out1_hbm.at[pl.ds(NS * RPT, REM_ROWS)])


@jax.jit
def kernel(x, edge_idx, edge_attr, W_rel, W_root, b):
  src = edge_idx[0]
  dst = edge_idx[1]
  idx2 = jnp.concatenate([2 * src, 2 * src + 1])
  b2 = b.reshape(L, 1, H)

  hr, root = _tc_first(x, W_rel[0], W_root[0], b2[0])
  for l in range(L):
    hr2 = hr.reshape(2 * N, HALF)
    a0, a1 = _sc_agg(hr2, idx2, dst, edge_attr)
    if l < L - 1:
      hr, root = _tc_step(a0, a1, root, W_rel[l + 1], W_root[l + 1], b2[l + 1])
    else:
      out = _tc_final(a0, a1, root)
  return out


# speedup vs baseline: 1.5554x; 1.0153x over previous
import functools

import jax
import jax.numpy as jnp
from jax import lax
from jax.experimental import pallas as pl
from jax.experimental.pallas import tpu as pltpu
from jax.experimental.pallas import tpu_sc as plsc

N = 10000
E = 160000
D = 256
H = 256
L = 3

NC = 2
NS = 16
HALF = H // NC
TPS = E // NS
CH = 40
NCHUNK = TPS // CH
NBUF = 5
NBLK = NCHUNK // NBUF
BE = NBUF * CH
LAG = 1
RPT = 624
REM_ROWS = N - NS * RPT

BN = 2000


def _tc_first_body(x_ref, wr_ref, wc_ref, b_ref, hr_ref, root_ref):
  h = x_ref[...]
  hr_ref[...] = jnp.dot(h, wr_ref[...], preferred_element_type=jnp.float32)
  root_ref[...] = (
      jnp.dot(h, wc_ref[...], preferred_element_type=jnp.float32) + b_ref[...])


def _tc_step_body(a0_ref, a1_ref, root_ref, wr_ref, wc_ref, b_ref,
                  hr_ref, rootn_ref):
  agg = jnp.concatenate([a0_ref[...], a1_ref[...]], axis=1)
  h = jnp.maximum(agg + root_ref[...], 0.0)
  hr_ref[...] = jnp.dot(h, wr_ref[...], preferred_element_type=jnp.float32)
  rootn_ref[...] = (
      jnp.dot(h, wc_ref[...], preferred_element_type=jnp.float32) + b_ref[...])


def _tc_final_body(a0_ref, a1_ref, root_ref, out_ref):
  agg = jnp.concatenate([a0_ref[...], a1_ref[...]], axis=1)
  out_ref[...] = jnp.maximum(agg + root_ref[...], 0.0)


def _row_spec(w):
  return pl.BlockSpec((BN, w), lambda i: (i, 0))


def _full_spec(h, w):
  return pl.BlockSpec((h, w), lambda i: (0, 0))


_tc_first = pl.pallas_call(
    _tc_first_body,
    grid=(N // BN,),
    in_specs=[_row_spec(D), _full_spec(D, H), _full_spec(D, H),
              _full_spec(1, H)],
    out_specs=[_row_spec(H), _row_spec(H)],
    out_shape=[jax.ShapeDtypeStruct((N, H), jnp.float32),
               jax.ShapeDtypeStruct((N, H), jnp.float32)],
)

_tc_step = pl.pallas_call(
    _tc_step_body,
    grid=(N // BN,),
    in_specs=[_row_spec(HALF), _row_spec(HALF), _row_spec(H),
              _full_spec(H, H), _full_spec(H, H), _full_spec(1, H)],
    out_specs=[_row_spec(H), _row_spec(H)],
    out_shape=[jax.ShapeDtypeStruct((N, H), jnp.float32),
               jax.ShapeDtypeStruct((N, H), jnp.float32)],
)

_tc_final = pl.pallas_call(
    _tc_final_body,
    grid=(N // BN,),
    in_specs=[_row_spec(HALF), _row_spec(HALF), _row_spec(H)],
    out_specs=_row_spec(H),
    out_shape=jax.ShapeDtypeStruct((N, H), jnp.float32),
)


_mesh = plsc.VectorSubcoreMesh(
    core_axis_name="c", subcore_axis_name="s", num_cores=NC, num_subcores=NS)


@functools.partial(
    pl.kernel,
    out_type=[jax.ShapeDtypeStruct((N, HALF), jnp.float32),
              jax.ShapeDtypeStruct((N, HALF), jnp.float32)],
    mesh=_mesh,
    scratch_types=(
        [pltpu.VMEM_SHARED((N, HALF), jnp.float32)]
        + [pltpu.VMEM((TPS,), jnp.int32)]
        + [pltpu.VMEM((TPS,), jnp.float32)]
        + [pltpu.VMEM((BE, HALF), jnp.float32)]
        + [pltpu.VMEM((CH,), jnp.int32)] * NBUF
        + [pltpu.SemaphoreType.DMA] * (3 * NBUF)
    ),
)
def _sc_agg(hr2_hbm, idx2_hbm, dst_hbm, w_hbm, out0_hbm, out1_hbm,
            acc, idx_all, w_all, rows, *rest):
  dstb = rest[:NBUF]
  gsems = rest[NBUF:2 * NBUF]
  ssems = rest[2 * NBUF:3 * NBUF]
  psems = rest[3 * NBUF:4 * NBUF]
  c = lax.axis_index("c")
  s = lax.axis_index("s")

  zero = jnp.zeros((16,), jnp.float32)

  def zrow(i, carry):
    for j in range(HALF // 16):
      rows[i, pl.ds(j * 16, 16)] = zero
    return carry

  lax.fori_loop(0, CH, zrow, 0)

  base = s * RPT
  nfull = RPT // CH
  for k in range(nfull):
    pltpu.sync_copy(rows.at[pl.ds(0, CH)], acc.at[pl.ds(base + k * CH, CH)])
  rem = RPT - nfull * CH
  pltpu.sync_copy(rows.at[pl.ds(0, rem)],
                  acc.at[pl.ds(base + nfull * CH, rem)])

  @pl.when(s == NS - 1)
  def _():
    pltpu.sync_copy(rows.at[pl.ds(0, REM_ROWS)],
                    acc.at[pl.ds(NS * RPT, REM_ROWS)])

  pltpu.sync_copy(idx2_hbm.at[pl.ds(c * E + s * TPS, TPS)], idx_all)
  pltpu.sync_copy(w_hbm.at[pl.ds(s * TPS, TPS)], w_all)

  plsc.subcore_barrier()

  def start_params(k, i):
    pltpu.async_copy(dst_hbm.at[pl.ds(s * TPS + i * CH, CH)], dstb[k],
                     psems[k])

  def wait_params(k):
    pltpu.make_async_copy(dst_hbm.at[pl.ds(0, CH)], dstb[k], psems[k]).wait()

  def slot(k):
    return rows.at[pl.ds(k * CH, CH)]

  def start_gather(k, i):
    pltpu.async_copy(hr2_hbm.at[idx_all.at[pl.ds(i * CH, CH)]],
                     slot(k), gsems[k])

  def wait_gather(k):
    pltpu.make_async_copy(hr2_hbm.at[idx_all.at[pl.ds(0, CH)]],
                          slot(k), gsems[k]).wait()

  def start_scatter(k):
    pltpu.async_copy(slot(k), acc.at[dstb[k]], ssems[k], add=True)

  def wait_scatter(k):
    pltpu.make_async_copy(slot(k), acc.at[dstb[k]], ssems[k]).wait()

  def scale_chunk(k, i):
    r0 = k * CH

    def body(g, carry):
      wvec = w_all[pl.ds(i * CH + g * 16, 16)]
      for u in range(16):
        e = r0 + g * 16 + u
        wv = wvec[u]
        for j in range(HALF // 16):
          sl = pl.ds(j * 16, 16)
          rows[e, sl] = rows[e, sl] * wv
      return carry

    lax.fori_loop(0, CH // 16, body, 0)
    ntail = CH % 16
    if ntail:
      wvec = w_all[pl.ds(i * CH + CH - 16, 16)]
      for u in range(ntail):
        e = r0 + CH - ntail + u
        wv = wvec[16 - ntail + u]
        for j in range(HALF // 16):
          sl = pl.ds(j * 16, 16)
          rows[e, sl] = rows[e, sl] * wv

  def drain_slot(j, g):
    wait_scatter(j)

    @pl.when(g < NBLK - 1)
    def _():
      i2 = (g + 1) * NBUF + j
      start_gather(j, i2)
      start_params(j, i2)

  for k in range(NBUF):
    start_params(k, k)
    start_gather(k, k)

  def gblock(g, carry):
    for k in range(NBUF):
      i = g * NBUF + k
      wait_gather(k)
      wait_params(k)
      scale_chunk(k, i)
      start_scatter(k)
      if k >= LAG:
        drain_slot(k - LAG, g)
    for j in range(NBUF - LAG, NBUF):
      drain_slot(j, g)
    return carry

  lax.fori_loop(0, NBLK, gblock, 0)

  plsc.subcore_barrier()

  @pl.when(c == 0)
  def _():
    pltpu.sync_copy(acc.at[pl.ds(base, RPT)], out0_hbm.at[pl.ds(base, RPT)])

    @pl.when(s == NS - 1)
    def _():
      pltpu.sync_copy(acc.at[pl.ds(NS * RPT, REM_ROWS)],
                      out0_hbm.at[pl.ds(NS * RPT, REM_ROWS)])

  @pl.when(c == 1)
  def _():
    pltpu.sync_copy(acc.at[pl.ds(base, RPT)], out1_hbm.at[pl.ds(base, RPT)])

    @pl.when(s == NS - 1)
    def _():
      pltpu.sync_copy(acc.at[pl.ds(NS * RPT, REM_ROWS)],
                      out1_hbm.at[pl.ds(NS * RPT, REM_ROWS)])


@jax.jit
def kernel(x, edge_idx, edge_attr, W_rel, W_root, b):
  src = edge_idx[0]
  dst = edge_idx[1]
  idx2 = jnp.concatenate([2 * src, 2 * src + 1])
  b2 = b.reshape(L, 1, H)

  hr, root = _tc_first(x, W_rel[0], W_root[0], b2[0])
  for l in range(L):
    hr2 = hr.reshape(2 * N, HALF)
    a0, a1 = _sc_agg(hr2, idx2, dst, edge_attr)
    if l < L - 1:
      hr, root = _tc_step(a0, a1, root, W_rel[l + 1], W_root[l + 1], b2[l + 1])
    else:
      out = _tc_final(a0, a1, root)
  return out


# speedup vs baseline: 1.5703x; 1.0095x over previous
import functools

import jax
import jax.numpy as jnp
from jax import lax
from jax.experimental import pallas as pl
from jax.experimental.pallas import tpu as pltpu
from jax.experimental.pallas import tpu_sc as plsc

N = 10000
E = 160000
D = 256
H = 256
L = 3

NC = 2
NS = 16
HALF = H // NC
TPS = E // NS
CH = 40
NCHUNK = TPS // CH
NBUF = 5
NBLK = NCHUNK // NBUF
BE = NBUF * CH
LAG = 1
RPT = 624
REM_ROWS = N - NS * RPT

BN = 5000


def _tc_first_body(x_ref, wr_ref, wc_ref, b_ref, hr_ref, root_ref):
  h = x_ref[...]
  hr_ref[...] = jnp.dot(h, wr_ref[...], preferred_element_type=jnp.float32)
  root_ref[...] = (
      jnp.dot(h, wc_ref[...], preferred_element_type=jnp.float32) + b_ref[...])


def _tc_step_body(a0_ref, a1_ref, root_ref, wr_ref, wc_ref, b_ref,
                  hr_ref, rootn_ref):
  agg = jnp.concatenate([a0_ref[...], a1_ref[...]], axis=1)
  h = jnp.maximum(agg + root_ref[...], 0.0)
  hr_ref[...] = jnp.dot(h, wr_ref[...], preferred_element_type=jnp.float32)
  rootn_ref[...] = (
      jnp.dot(h, wc_ref[...], preferred_element_type=jnp.float32) + b_ref[...])


def _tc_final_body(a0_ref, a1_ref, root_ref, out_ref):
  agg = jnp.concatenate([a0_ref[...], a1_ref[...]], axis=1)
  out_ref[...] = jnp.maximum(agg + root_ref[...], 0.0)


def _row_spec(w):
  return pl.BlockSpec((BN, w), lambda i: (i, 0))


def _full_spec(h, w):
  return pl.BlockSpec((h, w), lambda i: (0, 0))


_tc_first = pl.pallas_call(
    _tc_first_body,
    grid=(N // BN,),
    in_specs=[_row_spec(D), _full_spec(D, H), _full_spec(D, H),
              _full_spec(1, H)],
    out_specs=[_row_spec(H), _row_spec(H)],
    out_shape=[jax.ShapeDtypeStruct((N, H), jnp.float32),
               jax.ShapeDtypeStruct((N, H), jnp.float32)],
)

_tc_step = pl.pallas_call(
    _tc_step_body,
    grid=(N // BN,),
    in_specs=[_row_spec(HALF), _row_spec(HALF), _row_spec(H),
              _full_spec(H, H), _full_spec(H, H), _full_spec(1, H)],
    out_specs=[_row_spec(H), _row_spec(H)],
    out_shape=[jax.ShapeDtypeStruct((N, H), jnp.float32),
               jax.ShapeDtypeStruct((N, H), jnp.float32)],
)

_tc_final = pl.pallas_call(
    _tc_final_body,
    grid=(N // BN,),
    in_specs=[_row_spec(HALF), _row_spec(HALF), _row_spec(H)],
    out_specs=_row_spec(H),
    out_shape=jax.ShapeDtypeStruct((N, H), jnp.float32),
)


_mesh = plsc.VectorSubcoreMesh(
    core_axis_name="c", subcore_axis_name="s", num_cores=NC, num_subcores=NS)


@functools.partial(
    pl.kernel,
    out_type=[jax.ShapeDtypeStruct((N, HALF), jnp.float32),
              jax.ShapeDtypeStruct((N, HALF), jnp.float32)],
    mesh=_mesh,
    scratch_types=(
        [pltpu.VMEM_SHARED((N, HALF), jnp.float32)]
        + [pltpu.VMEM((TPS,), jnp.int32)]
        + [pltpu.VMEM((TPS,), jnp.float32)]
        + [pltpu.VMEM((BE, HALF), jnp.float32)]
        + [pltpu.VMEM((CH,), jnp.int32)] * NBUF
        + [pltpu.SemaphoreType.DMA] * (3 * NBUF)
    ),
)
def _sc_agg(hr2_hbm, idx2_hbm, dst_hbm, w_hbm, out0_hbm, out1_hbm,
            acc, idx_all, w_all, rows, *rest):
  dstb = rest[:NBUF]
  gsems = rest[NBUF:2 * NBUF]
  ssems = rest[2 * NBUF:3 * NBUF]
  psems = rest[3 * NBUF:4 * NBUF]
  c = lax.axis_index("c")
  s = lax.axis_index("s")

  zero = jnp.zeros((16,), jnp.float32)

  def zrow(i, carry):
    for j in range(HALF // 16):
      rows[i, pl.ds(j * 16, 16)] = zero
    return carry

  lax.fori_loop(0, CH, zrow, 0)

  base = s * RPT
  nfull = RPT // CH
  for k in range(nfull):
    pltpu.sync_copy(rows.at[pl.ds(0, CH)], acc.at[pl.ds(base + k * CH, CH)])
  rem = RPT - nfull * CH
  pltpu.sync_copy(rows.at[pl.ds(0, rem)],
                  acc.at[pl.ds(base + nfull * CH, rem)])

  @pl.when(s == NS - 1)
  def _():
    pltpu.sync_copy(rows.at[pl.ds(0, REM_ROWS)],
                    acc.at[pl.ds(NS * RPT, REM_ROWS)])

  pltpu.sync_copy(idx2_hbm.at[pl.ds(c * E + s * TPS, TPS)], idx_all)
  pltpu.sync_copy(w_hbm.at[pl.ds(s * TPS, TPS)], w_all)

  plsc.subcore_barrier()

  def start_params(k, i):
    pltpu.async_copy(dst_hbm.at[pl.ds(s * TPS + i * CH, CH)], dstb[k],
                     psems[k])

  def wait_params(k):
    pltpu.make_async_copy(dst_hbm.at[pl.ds(0, CH)], dstb[k], psems[k]).wait()

  def slot(k):
    return rows.at[pl.ds(k * CH, CH)]

  def start_gather(k, i):
    pltpu.async_copy(hr2_hbm.at[idx_all.at[pl.ds(i * CH, CH)]],
                     slot(k), gsems[k])

  def wait_gather(k):
    pltpu.make_async_copy(hr2_hbm.at[idx_all.at[pl.ds(0, CH)]],
                          slot(k), gsems[k]).wait()

  def start_scatter(k):
    pltpu.async_copy(slot(k), acc.at[dstb[k]], ssems[k], add=True)

  def wait_scatter(k):
    pltpu.make_async_copy(slot(k), acc.at[dstb[k]], ssems[k]).wait()

  def scale_chunk(k, i):
    r0 = k * CH

    def body(g, carry):
      wvec = w_all[pl.ds(i * CH + g * 16, 16)]
      for u in range(16):
        e = r0 + g * 16 + u
        wv = wvec[u]
        for j in range(HALF // 16):
          sl = pl.ds(j * 16, 16)
          rows[e, sl] = rows[e, sl] * wv
      return carry

    lax.fori_loop(0, CH // 16, body, 0)
    ntail = CH % 16
    if ntail:
      wvec = w_all[pl.ds(i * CH + CH - 16, 16)]
      for u in range(ntail):
        e = r0 + CH - ntail + u
        wv = wvec[16 - ntail + u]
        for j in range(HALF // 16):
          sl = pl.ds(j * 16, 16)
          rows[e, sl] = rows[e, sl] * wv

  def drain_slot(j, g):
    wait_scatter(j)

    @pl.when(g < NBLK - 1)
    def _():
      i2 = (g + 1) * NBUF + j
      start_gather(j, i2)
      start_params(j, i2)

  for k in range(NBUF):
    start_params(k, k)
    start_gather(k, k)

  def gblock(g, carry):
    for k in range(NBUF):
      i = g * NBUF + k
      wait_gather(k)
      wait_params(k)
      scale_chunk(k, i)
      start_scatter(k)
      if k >= LAG:
        drain_slot(k - LAG, g)
    for j in range(NBUF - LAG, NBUF):
      drain_slot(j, g)
    return carry

  lax.fori_loop(0, NBLK, gblock, 0)

  plsc.subcore_barrier()

  @pl.when(c == 0)
  def _():
    pltpu.sync_copy(acc.at[pl.ds(base, RPT)], out0_hbm.at[pl.ds(base, RPT)])

    @pl.when(s == NS - 1)
    def _():
      pltpu.sync_copy(acc.at[pl.ds(NS * RPT, REM_ROWS)],
                      out0_hbm.at[pl.ds(NS * RPT, REM_ROWS)])

  @pl.when(c == 1)
  def _():
    pltpu.sync_copy(acc.at[pl.ds(base, RPT)], out1_hbm.at[pl.ds(base, RPT)])

    @pl.when(s == NS - 1)
    def _():
      pltpu.sync_copy(acc.at[pl.ds(NS * RPT, REM_ROWS)],
                      out1_hbm.at[pl.ds(NS * RPT, REM_ROWS)])


@jax.jit
def kernel(x, edge_idx, edge_attr, W_rel, W_root, b):
  src = edge_idx[0]
  dst = edge_idx[1]
  idx2 = jnp.concatenate([2 * src, 2 * src + 1])
  b2 = b.reshape(L, 1, H)

  hr, root = _tc_first(x, W_rel[0], W_root[0], b2[0])
  for l in range(L):
    hr2 = hr.reshape(2 * N, HALF)
    a0, a1 = _sc_agg(hr2, idx2, dst, edge_attr)
    if l < L - 1:
      hr, root = _tc_step(a0, a1, root, W_rel[l + 1], W_root[l + 1], b2[l + 1])
    else:
      out = _tc_final(a0, a1, root)
  return out
